# baseline (speedup 1.0000x reference)
import functools

import jax
import jax.numpy as jnp
from jax import lax
from jax.experimental import pallas as pl


TB = 128


def _vq_block(z_ref, win_ref, bin_ref, ct_ref, cnorm_ref, wout_ref, bout_ref,
              zq_out_ref, idx_ref, loss_ref, *, n_codes):
    i = pl.program_id(0)

    z_e = lax.dot_general(z_ref[...], win_ref[...],
                          (((1,), (1,)), ((), ())),
                          preferred_element_type=jnp.float32)
    z_e = z_e + bin_ref[...]

    znorm = jnp.sum(z_e * z_e, axis=1, keepdims=True)

    zem2 = z_e * (-2.0)
    s = lax.dot_general(zem2, ct_ref[...],
                        (((1,), (0,)), ((), ())),
                        preferred_element_type=jnp.float32)
    d = (znorm + cnorm_ref[...]) + s

    minval = jnp.min(d, axis=1, keepdims=True)
    iota = lax.broadcasted_iota(jnp.int32, d.shape, 1)
    idx = jnp.min(jnp.where(d == minval, iota, n_codes),
                  axis=1, keepdims=True)
    idx_ref[...] = idx

    one_hot = jnp.where(iota == idx, 1.0, 0.0)
    z_q = lax.dot_general(one_hot, ct_ref[...],
                          (((1,), (1,)), ((), ())),
                          preferred_element_type=jnp.float32,
                          precision=lax.Precision.HIGHEST)

    zq_out_ref[...] = lax.dot_general(z_q, wout_ref[...],
                                      (((1,), (1,)), ((), ())),
                                      preferred_element_type=jnp.float32
                                      ) + bout_ref[...]

    diff = z_e - z_q
    part = jnp.sum(diff * diff).reshape(1, 1)

    @pl.when(i == 0)
    def _():
        loss_ref[...] = jnp.zeros_like(loss_ref)

    loss_ref[...] += part


def kernel(z, W_in, b_in, W_out, b_out, codebook):
    B, N, D = z.shape
    K, C = codebook.shape
    T = B * N
    nblk = T // TB

    z_flat = z.reshape(T, D)
    ct = codebook.T
    cnorm = jnp.sum(codebook ** 2, axis=-1)[None, :]

    zq_out, idx, loss_sum = pl.pallas_call(
        functools.partial(_vq_block, n_codes=K),
        grid=(nblk,),
        in_specs=[
            pl.BlockSpec((TB, D), lambda i: (i, 0)),
            pl.BlockSpec((C, D), lambda i: (0, 0)),
            pl.BlockSpec((1, C), lambda i: (0, 0)),
            pl.BlockSpec((C, K), lambda i: (0, 0)),
            pl.BlockSpec((1, K), lambda i: (0, 0)),
            pl.BlockSpec((D, C), lambda i: (0, 0)),
            pl.BlockSpec((1, D), lambda i: (0, 0)),
        ],
        out_specs=[
            pl.BlockSpec((TB, D), lambda i: (i, 0)),
            pl.BlockSpec((TB, 1), lambda i: (i, 0)),
            pl.BlockSpec((1, 1), lambda i: (0, 0)),
        ],
        out_shape=[
            jax.ShapeDtypeStruct((T, D), jnp.float32),
            jax.ShapeDtypeStruct((T, 1), jnp.int32),
            jax.ShapeDtypeStruct((1, 1), jnp.float32),
        ],
    )(z_flat, W_in, b_in.reshape(1, C), ct, cnorm, W_out, b_out.reshape(1, D))

    z_q_out = zq_out.reshape(B, N, D)
    indices = idx.reshape(B, N)
    loss = loss_sum[0, 0] / (T * C)
    return (z_q_out, indices, loss, loss)
